# baseline (speedup 1.0000x reference)
import jax
import jax.numpy as jnp
from jax import lax
from jax.experimental import pallas as pl
from jax.experimental.pallas import tpu as pltpu
from jax.experimental.pallas import tpu_sc as plsc

BATCH = 16384
EMBED_DIM = 16
BUCKETS = 100000
NUM_CORES = 2
NUM_SUBCORES = 16
NUM_WORKERS = NUM_CORES * NUM_SUBCORES
B_PER_W = BATCH // NUM_WORKERS
LANES = 16


def _fm_body(uid_hbm, tid_hbm, utab_hbm, itab_hbm, wb_hbm, out_hbm,
             idx_u_v, idx_t_v, rows_u_v, rows_t_v, out_v, wb_v,
             sem_u, sem_t):
    wid = lax.axis_index("s") * NUM_CORES + lax.axis_index("c")
    base = wid * B_PER_W

    pltpu.sync_copy(uid_hbm.at[pl.ds(base, B_PER_W)], idx_u_v)
    pltpu.sync_copy(tid_hbm.at[pl.ds(base, B_PER_W)], idx_t_v)
    cu = pltpu.async_copy(utab_hbm.at[idx_u_v], rows_u_v, sem_u)
    ct = pltpu.async_copy(itab_hbm.at[idx_t_v], rows_t_v, sem_t)
    pltpu.sync_copy(wb_hbm, wb_v)
    cu.wait()
    ct.wait()

    wv = wb_v[pl.ds(0, LANES)]
    bv = wb_v[pl.ds(LANES, LANES)]
    lanes = lax.iota(jnp.int32, LANES)

    def grp(g, carry):
        row = g * LANES + lanes
        acc = jnp.zeros((LANES,), jnp.float32)
        for d in range(EMBED_DIM):
            col = jnp.full((LANES,), d, jnp.int32)
            cu_ = plsc.load_gather(rows_u_v, [row, col])
            ct_ = plsc.load_gather(rows_t_v, [row, col])
            acc = acc + cu_ * ct_
        z = acc * wv + bv
        y = 1.0 / (1.0 + jnp.exp(-z))
        out_v[pl.ds(g * LANES, LANES)] = y
        return carry

    lax.fori_loop(0, B_PER_W // LANES, grp, 0)

    pltpu.sync_copy(out_v, out_hbm.at[pl.ds(base, B_PER_W)])


@jax.jit
def _fm_sc(f_uid, f_tid, utab_rm, itab_rm, wb):
    mesh = plsc.VectorSubcoreMesh(core_axis_name="c", subcore_axis_name="s")
    return pl.kernel(
        _fm_body,
        out_type=jax.ShapeDtypeStruct((BATCH,), jnp.float32),
        mesh=mesh,
        compiler_params=pltpu.CompilerParams(
            needs_layout_passes=False, use_tc_tiling_on_sc=False),
        scratch_types=[
            pltpu.VMEM((B_PER_W,), jnp.int32),
            pltpu.VMEM((B_PER_W,), jnp.int32),
            pltpu.VMEM((B_PER_W, EMBED_DIM), jnp.float32),
            pltpu.VMEM((B_PER_W, EMBED_DIM), jnp.float32),
            pltpu.VMEM((B_PER_W,), jnp.float32),
            pltpu.VMEM((8 * LANES,), jnp.float32),
            pltpu.SemaphoreType.DMA,
            pltpu.SemaphoreType.DMA,
        ],
    )(f_uid, f_tid, utab_rm, itab_rm, wb)


def _row_major(table):
    flat = lax.optimization_barrier(table.reshape(-1))
    return flat.reshape(BUCKETS, EMBED_DIM)


def kernel(f_uid, f_tid, user_table, item_table, W, b):
    wb = jnp.concatenate([
        jnp.broadcast_to(W.reshape(1), (LANES,)),
        jnp.broadcast_to(b.reshape(1), (LANES,)),
        jnp.zeros((8 * LANES - 2 * LANES,), jnp.float32),
    ])
    y = _fm_sc(f_uid, f_tid, _row_major(user_table), _row_major(item_table), wb)
    return y.reshape(BATCH, 1)


# speedup vs baseline: 1.9017x; 1.9017x over previous
import jax
import jax.numpy as jnp
from jax import lax
from jax.experimental import pallas as pl
from jax.experimental.pallas import tpu as pltpu
from jax.experimental.pallas import tpu_sc as plsc

BATCH = 16384
EMBED_DIM = 16
BUCKETS = 100000
NUM_CORES = 2
NUM_SUBCORES = 16
NUM_WORKERS = NUM_CORES * NUM_SUBCORES
B_PER_W = BATCH // NUM_WORKERS
LANES = 16
NWORDS = B_PER_W * EMBED_DIM
NCHUNK = 4
C_ROWS = B_PER_W // NCHUNK
C_WORDS = C_ROWS * EMBED_DIM


def _worker_id():
    return lax.axis_index("s") * NUM_CORES + lax.axis_index("c")


def _build_fid(idx_v, fid_v, c):
    def body(j, carry):
        iv = idx_v[pl.ds(c * C_ROWS + j * LANES, LANES)]
        for d in range(EMBED_DIM):
            s = pl.ds(c * C_WORDS + d * C_ROWS + j * LANES, LANES)
            fid_v[s] = iv + (d * BUCKETS)
        return carry
    lax.fori_loop(0, C_ROWS // LANES, body, 0)


def _gather_u_body(uid_hbm, utab_hbm, gu_hbm,
                   idx_v, fid_v, g_v, s0, s1, s2, s3):
    sems = (s0, s1, s2, s3)
    base = _worker_id() * B_PER_W
    pltpu.sync_copy(uid_hbm.at[pl.ds(base, B_PER_W)], idx_v)
    copies = []
    for c in range(NCHUNK):
        _build_fid(idx_v, fid_v, c)
        sl = pl.ds(c * C_WORDS, C_WORDS)
        copies.append(
            pltpu.async_copy(utab_hbm.at[fid_v.at[sl]], g_v.at[sl], sems[c]))
    for cp in copies:
        cp.wait()
    pltpu.sync_copy(g_v, gu_hbm.at[pl.ds(base * EMBED_DIM, NWORDS)])


def _gather_t_compute_body(tid_hbm, itab_hbm, gu_hbm, wb_hbm, out_hbm,
                           idx_v, fid_v, gu_v, gt_v, out_v, wb_v,
                           s0, s1, s2, s3, sg):
    sems = (s0, s1, s2, s3)
    wid = _worker_id()
    base = wid * B_PER_W
    pltpu.sync_copy(tid_hbm.at[pl.ds(base, B_PER_W)], idx_v)
    cg = pltpu.async_copy(gu_hbm.at[pl.ds(base * EMBED_DIM, NWORDS)], gu_v, sg)
    copies = []
    for c in range(NCHUNK):
        _build_fid(idx_v, fid_v, c)
        sl = pl.ds(c * C_WORDS, C_WORDS)
        copies.append(
            pltpu.async_copy(itab_hbm.at[fid_v.at[sl]], gt_v.at[sl], sems[c]))
    pltpu.sync_copy(wb_hbm, wb_v)
    wv = wb_v[pl.ds(0, LANES)]
    bv = wb_v[pl.ds(LANES, LANES)]
    cg.wait()
    for c in range(NCHUNK):
        copies[c].wait()

        def grp(j, carry):
            acc = jnp.zeros((LANES,), jnp.float32)
            for d in range(EMBED_DIM):
                s = pl.ds(c * C_WORDS + d * C_ROWS + j * LANES, LANES)
                acc = acc + gu_v[s] * gt_v[s]
            z = acc * wv + bv
            y = 1.0 / (1.0 + jnp.exp(-z))
            out_v[pl.ds(c * C_ROWS + j * LANES, LANES)] = y
            return carry

        lax.fori_loop(0, C_ROWS // LANES, grp, 0)
    pltpu.sync_copy(out_v, out_hbm.at[pl.ds(base, B_PER_W)])


@jax.jit
def _fm_sc(f_uid, f_tid, utab_flat, itab_flat, wb):
    mesh = plsc.VectorSubcoreMesh(core_axis_name="c", subcore_axis_name="s")
    params = pltpu.CompilerParams(needs_layout_passes=False)
    gu = pl.kernel(
        _gather_u_body,
        out_type=jax.ShapeDtypeStruct((BATCH * EMBED_DIM,), jnp.float32),
        mesh=mesh,
        compiler_params=params,
        scratch_types=[
            pltpu.VMEM((B_PER_W,), jnp.int32),
            pltpu.VMEM((NWORDS,), jnp.int32),
            pltpu.VMEM((NWORDS,), jnp.float32),
            pltpu.SemaphoreType.DMA,
            pltpu.SemaphoreType.DMA,
            pltpu.SemaphoreType.DMA,
            pltpu.SemaphoreType.DMA,
        ],
    )(f_uid, utab_flat)
    return pl.kernel(
        _gather_t_compute_body,
        out_type=jax.ShapeDtypeStruct((BATCH,), jnp.float32),
        mesh=mesh,
        compiler_params=params,
        scratch_types=[
            pltpu.VMEM((B_PER_W,), jnp.int32),
            pltpu.VMEM((NWORDS,), jnp.int32),
            pltpu.VMEM((NWORDS,), jnp.float32),
            pltpu.VMEM((NWORDS,), jnp.float32),
            pltpu.VMEM((B_PER_W,), jnp.float32),
            pltpu.VMEM((8 * LANES,), jnp.float32),
            pltpu.SemaphoreType.DMA,
            pltpu.SemaphoreType.DMA,
            pltpu.SemaphoreType.DMA,
            pltpu.SemaphoreType.DMA,
            pltpu.SemaphoreType.DMA,
        ],
    )(f_tid, itab_flat, gu, wb)


def kernel(f_uid, f_tid, user_table, item_table, W, b):
    utab_flat = user_table.T.reshape(-1)
    itab_flat = item_table.T.reshape(-1)
    wb = jnp.concatenate([
        jnp.broadcast_to(W.reshape(1), (LANES,)),
        jnp.broadcast_to(b.reshape(1), (LANES,)),
        jnp.zeros((8 * LANES - 2 * LANES,), jnp.float32),
    ])
    y = _fm_sc(f_uid, f_tid, utab_flat, itab_flat, wb)
    return y.reshape(BATCH, 1)
